# Initial kernel scaffold; baseline (speedup 1.0000x reference)
import jax
import jax.numpy as jnp
from jax.experimental import pallas as pl


def kernel(x, edge_index, W1, b1, W2, b2, LW1, Lb1, LW2, Lb2):
    raise NotImplementedError("write your pallas kernel here")


# speedup vs baseline: 5.1473x; 5.1473x over previous
import functools

import jax
import jax.numpy as jnp
from jax import lax
from jax.experimental import pallas as pl
from jax.experimental.pallas import tpu as pltpu
from jax.experimental.pallas import tpu_sc as plsc

N = 10000
E = 320000
D = 128

NC = 2
NS = 16
NW = NC * NS
K = 128

NPAD = ((N + NW * 8 - 1) // (NW * 8)) * NW * 8
RPS = NPAD // NS
CHUNKS = ((-(-E // (NW * K)) + 7) // 8) * 8
EPT = CHUNKS * K
EPAD = NW * EPT

_mesh = plsc.VectorSubcoreMesh(core_axis_name="c", subcore_axis_name="s")


def _wid():
    return lax.axis_index("c") * NS + lax.axis_index("s")


@functools.partial(
    pl.kernel,
    out_type=jax.ShapeDtypeStruct((NC * NPAD,), jnp.float32),
    mesh=_mesh,
    scratch_types=[
        pltpu.VMEM_SHARED((NPAD,), jnp.float32),
        pltpu.VMEM((CHUNKS, K), jnp.int32),
        pltpu.VMEM((K,), jnp.float32),
    ],
)
def _sc_degree(col2d, zeros1, deg_out, acc, call, ones):
    core = lax.axis_index("c")
    sid = lax.axis_index("s")
    w = _wid()
    pltpu.sync_copy(zeros1.at[pl.ds(sid * RPS, RPS)], acc.at[pl.ds(sid * RPS, RPS)])
    pltpu.sync_copy(col2d.at[pl.ds(w * CHUNKS, CHUNKS)], call)
    for j in range(K // 16):
        ones[pl.ds(j * 16, 16)] = jnp.ones((16,), jnp.float32)
    plsc.subcore_barrier()

    def step(i, c):
        pltpu.sync_copy(ones, acc.at[call.at[i]], add=True)
        return c

    lax.fori_loop(0, CHUNKS, step, 0)
    plsc.subcore_barrier()
    pltpu.sync_copy(
        acc.at[pl.ds(sid * RPS, RPS)],
        deg_out.at[pl.ds(core * NPAD + sid * RPS, RPS)],
    )


@functools.partial(
    pl.kernel,
    out_type=jax.ShapeDtypeStruct((NC * NPAD, D), jnp.float32),
    mesh=_mesh,
    scratch_types=[
        pltpu.VMEM_SHARED((NPAD, D), jnp.float32),
        pltpu.VMEM((CHUNKS, K), jnp.int32),
        pltpu.VMEM((CHUNKS, K), jnp.int32),
        pltpu.VMEM((K, D), jnp.float32),
        pltpu.SemaphoreType.DMA,
    ],
)
def _sc_scatter(hs, row2d, col2d, zeros2, acc_out, acc, rall, call, msg, sem):
    core = lax.axis_index("c")
    sid = lax.axis_index("s")
    w = _wid()
    pltpu.sync_copy(zeros2.at[pl.ds(sid * RPS, RPS)], acc.at[pl.ds(sid * RPS, RPS)])
    pltpu.sync_copy(row2d.at[pl.ds(w * CHUNKS, CHUNKS)], rall)
    pltpu.sync_copy(col2d.at[pl.ds(w * CHUNKS, CHUNKS)], call)
    plsc.subcore_barrier()

    def step(i, c):
        pltpu.async_copy(hs.at[rall.at[i]], msg, sem).wait()
        pltpu.sync_copy(msg, acc.at[call.at[i]], add=True)
        return c

    lax.fori_loop(0, CHUNKS, step, 0)
    plsc.subcore_barrier()
    pltpu.sync_copy(
        acc.at[pl.ds(sid * RPS, RPS)],
        acc_out.at[pl.ds(core * NPAD + sid * RPS, RPS)],
    )


@functools.partial(
    pl.kernel,
    out_type=jax.ShapeDtypeStruct((EPAD, 16), jnp.float32),
    mesh=_mesh,
    scratch_types=[
        pltpu.VMEM((CHUNKS, K), jnp.int32),
        pltpu.VMEM((CHUNKS, K), jnp.int32),
        pltpu.VMEM((K, D), jnp.float32),
        pltpu.VMEM((K, D), jnp.float32),
        pltpu.VMEM((K, 16), jnp.float32),
        pltpu.VMEM((D,), jnp.float32),
        pltpu.SemaphoreType.DMA,
        pltpu.SemaphoreType.DMA,
    ],
)
def _sc_decoder(u, row2d, col2d, w2, out, rall, call, bs, bd, ob, wv, sem, sem2):
    w = _wid()
    pltpu.sync_copy(row2d.at[pl.ds(w * CHUNKS, CHUNKS)], rall)
    pltpu.sync_copy(col2d.at[pl.ds(w * CHUNKS, CHUNKS)], call)
    pltpu.sync_copy(w2, wv)
    wregs = [wv[pl.ds(j * 16, 16)] for j in range(D // 16)]

    def chunk(i, c):
        ga = pltpu.async_copy(u.at[rall.at[i]], bs, sem)
        gb = pltpu.async_copy(u.at[call.at[i]], bd, sem2)
        ga.wait()
        gb.wait()

        def edge(e, c2):
            vacc = jnp.zeros((16,), jnp.float32)
            for j in range(D // 16):
                gg = jnp.maximum(
                    bs[e, pl.ds(j * 16, 16)] + bd[e, pl.ds(j * 16, 16)], 0.0
                )
                vacc = vacc + gg * wregs[j]
            ob[e] = vacc
            return c2

        lax.fori_loop(0, K, edge, 0)
        pltpu.sync_copy(ob, out.at[pl.ds(w * EPT + i * K, K)])
        return c

    lax.fori_loop(0, CHUNKS, chunk, 0)


BR = 2560


def _tc_a_body(x_ref, w_ref, degp_ref, hs_ref, dinv_ref):
    deg = degp_ref[0] + degp_ref[1] + 1.0
    dinv = lax.rsqrt(deg)
    dinv_ref[...] = dinv
    hs_ref[...] = jnp.dot(x_ref[...], w_ref[...], preferred_element_type=jnp.float32) * dinv


_tc_a = pl.pallas_call(
    _tc_a_body,
    grid=(NPAD // BR,),
    in_specs=[
        pl.BlockSpec((BR, D), lambda i: (i, 0)),
        pl.BlockSpec((D, D), lambda i: (0, 0)),
        pl.BlockSpec((2, BR, 1), lambda i: (0, i, 0)),
    ],
    out_specs=[
        pl.BlockSpec((BR, D), lambda i: (i, 0)),
        pl.BlockSpec((BR, 1), lambda i: (i, 0)),
    ],
    out_shape=[
        jax.ShapeDtypeStruct((NPAD, D), jnp.float32),
        jax.ShapeDtypeStruct((NPAD, 1), jnp.float32),
    ],
)


def _tc_b_body(acc_ref, hs_ref, dinv_ref, b_ref, w_ref, out_ref):
    z = (acc_ref[0] + acc_ref[1] + hs_ref[...]) * dinv_ref[...] + b_ref[...]
    z = jnp.maximum(z, 0.0)
    out_ref[...] = jnp.dot(z, w_ref[...], preferred_element_type=jnp.float32) * dinv_ref[...]


_tc_b = pl.pallas_call(
    _tc_b_body,
    grid=(NPAD // BR,),
    in_specs=[
        pl.BlockSpec((2, BR, D), lambda i: (0, i, 0)),
        pl.BlockSpec((BR, D), lambda i: (i, 0)),
        pl.BlockSpec((BR, 1), lambda i: (i, 0)),
        pl.BlockSpec((1, D), lambda i: (0, 0)),
        pl.BlockSpec((D, D), lambda i: (0, 0)),
    ],
    out_specs=pl.BlockSpec((BR, D), lambda i: (i, 0)),
    out_shape=jax.ShapeDtypeStruct((NPAD, D), jnp.float32),
)


def _tc_c_body(acc_ref, hs_ref, dinv_ref, b_ref, w_ref, lb_ref, u_ref):
    z = (acc_ref[0] + acc_ref[1] + hs_ref[...]) * dinv_ref[...] + b_ref[...]
    u_ref[...] = (
        jnp.dot(z, w_ref[...], preferred_element_type=jnp.float32) * 0.5
        + 0.5 * lb_ref[...]
    )


_tc_c = pl.pallas_call(
    _tc_c_body,
    grid=(NPAD // BR,),
    in_specs=[
        pl.BlockSpec((2, BR, D), lambda i: (0, i, 0)),
        pl.BlockSpec((BR, D), lambda i: (i, 0)),
        pl.BlockSpec((BR, 1), lambda i: (i, 0)),
        pl.BlockSpec((1, D), lambda i: (0, 0)),
        pl.BlockSpec((D, D), lambda i: (0, 0)),
        pl.BlockSpec((1, D), lambda i: (0, 0)),
    ],
    out_specs=pl.BlockSpec((BR, D), lambda i: (i, 0)),
    out_shape=jax.ShapeDtypeStruct((NPAD, D), jnp.float32),
)


def _tc_d_body(p_ref, lb_ref, out_ref):
    out_ref[...] = jnp.sum(p_ref[...], axis=1, keepdims=True) + lb_ref[...]


BE = 8192

_tc_d = pl.pallas_call(
    _tc_d_body,
    grid=(EPAD // BE,),
    in_specs=[
        pl.BlockSpec((BE, 16), lambda i: (i, 0)),
        pl.BlockSpec((1, 1), lambda i: (0, 0)),
    ],
    out_specs=pl.BlockSpec((BE, 1), lambda i: (i, 0)),
    out_shape=jax.ShapeDtypeStruct((EPAD, 1), jnp.float32),
)


def kernel(x, edge_index, W1, b1, W2, b2, LW1, Lb1, LW2, Lb2):
    pad_e = EPAD - E
    rowp = jnp.concatenate([edge_index[0], jnp.zeros((pad_e,), jnp.int32)])
    colp = jnp.concatenate([edge_index[1], jnp.full((pad_e,), N, jnp.int32)])
    row2d = rowp.reshape(EPAD // K, K)
    col2d = colp.reshape(EPAD // K, K)
    xp = jnp.pad(x, ((0, NPAD - N), (0, 0)))
    zeros1 = jnp.zeros((NPAD,), jnp.float32)
    zeros2 = jnp.zeros((NPAD, D), jnp.float32)

    degp = _sc_degree(col2d, zeros1).reshape(2, NPAD, 1)
    hs1, dinv = _tc_a(xp, W1, degp)
    acc1 = _sc_scatter(hs1, row2d, col2d, zeros2).reshape(2, NPAD, D)
    hs2 = _tc_b(acc1, hs1, dinv, b1.reshape(1, D), W2)
    acc2 = _sc_scatter(hs2, row2d, col2d, zeros2).reshape(2, NPAD, D)
    u = _tc_c(acc2, hs2, dinv, b2.reshape(1, D), LW1, Lb1.reshape(1, D))
    parts = _sc_decoder(u, row2d, col2d, LW2.reshape(D))
    out = _tc_d(parts, Lb2.reshape(1, 1))
    return out.reshape(EPAD)[:E]


# speedup vs baseline: 5.6876x; 1.1050x over previous
import functools

import jax
import jax.numpy as jnp
from jax import lax
from jax.experimental import pallas as pl
from jax.experimental.pallas import tpu as pltpu
from jax.experimental.pallas import tpu_sc as plsc

N = 10000
E = 320000
D = 128

NC = 2
NS = 16
NW = NC * NS
K = 128

NPAD = ((N + NW * 8 - 1) // (NW * 8)) * NW * 8
RPS = NPAD // NS
CHUNKS = ((-(-E // (NW * K)) + 7) // 8) * 8
EPT = CHUNKS * K
EPAD = NW * EPT
NCHUNKS = NW * CHUNKS

C0 = 80
C1 = 80
CMAX = max(C0, C1)
IDXROWS = ((NCHUNKS + CMAX + 7) // 8) * 8

_mesh = plsc.VectorSubcoreMesh(core_axis_name="c", subcore_axis_name="s")


def _wid():
    return lax.axis_index("c") * NS + lax.axis_index("s")


def _core_span():
    core = lax.axis_index("c")
    sid = lax.axis_index("s")
    base = jnp.where(core == 0, sid * C0, NS * C0 + sid * C1)
    nch = jnp.where(core == 0, C0, C1)
    return base, nch


@functools.partial(
    pl.kernel,
    out_type=jax.ShapeDtypeStruct((NC * NPAD,), jnp.float32),
    mesh=_mesh,
    scratch_types=[
        pltpu.VMEM_SHARED((NPAD,), jnp.float32),
        pltpu.VMEM((CHUNKS, K), jnp.int32),
        pltpu.VMEM((K,), jnp.float32),
    ],
)
def _sc_degree(col2d, zeros1, deg_out, acc, call, ones):
    core = lax.axis_index("c")
    sid = lax.axis_index("s")
    w = _wid()
    pltpu.sync_copy(zeros1.at[pl.ds(sid * RPS, RPS)], acc.at[pl.ds(sid * RPS, RPS)])
    pltpu.sync_copy(col2d.at[pl.ds(w * CHUNKS, CHUNKS)], call)
    for j in range(K // 16):
        ones[pl.ds(j * 16, 16)] = jnp.ones((16,), jnp.float32)
    plsc.subcore_barrier()

    def step(i, c):
        pltpu.sync_copy(ones, acc.at[call.at[i]], add=True)
        return c

    lax.fori_loop(0, CHUNKS, step, 0)
    plsc.subcore_barrier()
    pltpu.sync_copy(
        acc.at[pl.ds(sid * RPS, RPS)],
        deg_out.at[pl.ds(core * NPAD + sid * RPS, RPS)],
    )


@functools.partial(
    pl.kernel,
    out_type=jax.ShapeDtypeStruct((NC * NPAD, D), jnp.float32),
    mesh=_mesh,
    scratch_types=[
        pltpu.VMEM_SHARED((NPAD, D), jnp.float32),
        pltpu.VMEM((CMAX // 2, K), jnp.int32),
        pltpu.VMEM((CMAX // 2, K), jnp.int32),
        pltpu.VMEM((K, D), jnp.float32),
        pltpu.VMEM((K, D), jnp.float32),
        pltpu.SemaphoreType.DMA,
        pltpu.SemaphoreType.DMA,
    ],
)
def _sc_scatter(hs, row2d, col2d, zeros2, acc_out, acc, rall, call, msg0, msg1, sem0, sem1):
    core = lax.axis_index("c")
    sid = lax.axis_index("s")
    base, nch = _core_span()
    hc = nch // 2
    pltpu.sync_copy(zeros2.at[pl.ds(sid * RPS, RPS)], acc.at[pl.ds(sid * RPS, RPS)])
    plsc.subcore_barrier()

    for h in range(2):
        off = pl.multiple_of(base + h * hc, 8)
        pltpu.sync_copy(row2d.at[pl.ds(off, CMAX // 2)], rall)
        pltpu.sync_copy(col2d.at[pl.ds(off, CMAX // 2)], call)

        pltpu.async_copy(hs.at[rall.at[0]], msg0, sem0)

        def step(g, c):
            i0 = 2 * g
            i1 = i0 + 1
            i2 = jnp.minimum(i1 + 1, hc - 1)
            pltpu.make_async_copy(hs.at[rall.at[i0]], msg0, sem0).wait()
            pltpu.async_copy(hs.at[rall.at[i1]], msg1, sem1)
            pltpu.sync_copy(msg0, acc.at[call.at[i0]], add=True)
            pltpu.make_async_copy(hs.at[rall.at[i1]], msg1, sem1).wait()
            pltpu.async_copy(hs.at[rall.at[i2]], msg0, sem0)
            pltpu.sync_copy(msg1, acc.at[call.at[i1]], add=True)
            return c

        lax.fori_loop(0, hc // 2, step, 0)
        pltpu.make_async_copy(hs.at[rall.at[0]], msg0, sem0).wait()

    plsc.subcore_barrier()
    pltpu.sync_copy(
        acc.at[pl.ds(sid * RPS, RPS)],
        acc_out.at[pl.ds(core * NPAD + sid * RPS, RPS)],
    )


@functools.partial(
    pl.kernel,
    out_type=jax.ShapeDtypeStruct((EPAD, D), jnp.float32),
    mesh=_mesh,
    scratch_types=[
        pltpu.VMEM((CMAX, K), jnp.int32),
        pltpu.VMEM((CMAX, K), jnp.int32),
        pltpu.VMEM((K, D), jnp.float32),
        pltpu.VMEM((K, D), jnp.float32),
        pltpu.VMEM((K, D), jnp.float32),
        pltpu.VMEM((K, D), jnp.float32),
        pltpu.VMEM((K, D), jnp.float32),
        pltpu.VMEM((K, D), jnp.float32),
        pltpu.SemaphoreType.DMA,
        pltpu.SemaphoreType.DMA,
        pltpu.SemaphoreType.DMA,
        pltpu.SemaphoreType.DMA,
    ],
)
def _sc_edge(z, row2d, col2d, out, rall, call, bs0, bd0, bs1, bd1, ob0, ob1, semA, semB, semO0, semO1):
    base, nch = _core_span()
    base = pl.multiple_of(base, 8)
    ebase = pl.multiple_of(base * K, 8)
    pltpu.sync_copy(row2d.at[pl.ds(base, CMAX)], rall)
    pltpu.sync_copy(col2d.at[pl.ds(base, CMAX)], call)

    def issue(i, bs, bd, sem):
        pltpu.async_copy(z.at[rall.at[i]], bs, sem)
        pltpu.async_copy(z.at[call.at[i]], bd, sem)

    def drain(bs, bd, sem):
        pltpu.make_async_copy(z.at[rall.at[0]], bs, sem).wait()
        pltpu.make_async_copy(z.at[call.at[0]], bd, sem).wait()

    def compute(i, bs, bd, obx, semO, g):
        @pl.when(g > 0)
        def _():
            pltpu.make_async_copy(obx, out.at[pl.ds(ebase, K)], semO).wait()

        def edge(e, c2):
            for j in range(D // 16):
                obx[e, pl.ds(j * 16, 16)] = (
                    bs[e, pl.ds(j * 16, 16)] + bd[e, pl.ds(j * 16, 16)]
                ) * 0.5
            return c2

        lax.fori_loop(0, K, edge, 0)
        pltpu.async_copy(obx, out.at[pl.ds(pl.multiple_of(ebase + i * K, 8), K)], semO)

    issue(0, bs0, bd0, semA)

    def chunk(g, c):
        i0 = 2 * g
        i1 = i0 + 1
        i2 = jnp.minimum(i1 + 1, nch - 1)
        drain(bs0, bd0, semA)
        issue(i1, bs1, bd1, semB)
        compute(i0, bs0, bd0, ob0, semO0, g)
        drain(bs1, bd1, semB)
        issue(i2, bs0, bd0, semA)
        compute(i1, bs1, bd1, ob1, semO1, g)
        return c

    lax.fori_loop(0, nch // 2, chunk, 0)
    drain(bs0, bd0, semA)
    pltpu.make_async_copy(ob0, out.at[pl.ds(ebase, K)], semO0).wait()
    pltpu.make_async_copy(ob1, out.at[pl.ds(ebase, K)], semO1).wait()


BR = 2560


def _bf16_dot(a, b):
    return jnp.dot(
        a.astype(jnp.bfloat16),
        b.astype(jnp.bfloat16),
        preferred_element_type=jnp.float32,
    )


def _dot3x(a, b):
    ah = a.astype(jnp.bfloat16)
    bh = b.astype(jnp.bfloat16)
    al = (a - ah.astype(jnp.float32)).astype(jnp.bfloat16)
    bl = (b - bh.astype(jnp.float32)).astype(jnp.bfloat16)

    def d(p, q):
        return jnp.dot(p, q, preferred_element_type=jnp.float32)

    return d(ah, bh) + (d(ah, bl) + d(al, bh))


def _tc_a_body(x_ref, w_ref, degp_ref, hs_ref, dinv_ref):
    deg = degp_ref[0] + degp_ref[1] + 1.0
    dinv = lax.rsqrt(deg)
    dinv_ref[...] = dinv
    hs_ref[...] = _bf16_dot(x_ref[...], w_ref[...]) * dinv


_tc_a = pl.pallas_call(
    _tc_a_body,
    grid=(NPAD // BR,),
    in_specs=[
        pl.BlockSpec((BR, D), lambda i: (i, 0)),
        pl.BlockSpec((D, D), lambda i: (0, 0)),
        pl.BlockSpec((2, BR, 1), lambda i: (0, i, 0)),
    ],
    out_specs=[
        pl.BlockSpec((BR, D), lambda i: (i, 0)),
        pl.BlockSpec((BR, 1), lambda i: (i, 0)),
    ],
    out_shape=[
        jax.ShapeDtypeStruct((NPAD, D), jnp.float32),
        jax.ShapeDtypeStruct((NPAD, 1), jnp.float32),
    ],
)


def _tc_b_body(acc_ref, hs_ref, dinv_ref, b_ref, w_ref, out_ref):
    z = (acc_ref[0] + acc_ref[1] + hs_ref[...]) * dinv_ref[...] + b_ref[...]
    z = jnp.maximum(z, 0.0)
    out_ref[...] = _bf16_dot(z, w_ref[...]) * dinv_ref[...]


_tc_b = pl.pallas_call(
    _tc_b_body,
    grid=(NPAD // BR,),
    in_specs=[
        pl.BlockSpec((2, BR, D), lambda i: (0, i, 0)),
        pl.BlockSpec((BR, D), lambda i: (i, 0)),
        pl.BlockSpec((BR, 1), lambda i: (i, 0)),
        pl.BlockSpec((1, D), lambda i: (0, 0)),
        pl.BlockSpec((D, D), lambda i: (0, 0)),
    ],
    out_specs=pl.BlockSpec((BR, D), lambda i: (i, 0)),
    out_shape=jax.ShapeDtypeStruct((NPAD, D), jnp.float32),
)


def _tc_c_body(acc_ref, hs_ref, dinv_ref, b_ref, z_ref):
    z_ref[...] = (acc_ref[0] + acc_ref[1] + hs_ref[...]) * dinv_ref[...] + b_ref[...]


_tc_c = pl.pallas_call(
    _tc_c_body,
    grid=(NPAD // BR,),
    in_specs=[
        pl.BlockSpec((2, BR, D), lambda i: (0, i, 0)),
        pl.BlockSpec((BR, D), lambda i: (i, 0)),
        pl.BlockSpec((BR, 1), lambda i: (i, 0)),
        pl.BlockSpec((1, D), lambda i: (0, 0)),
    ],
    out_specs=pl.BlockSpec((BR, D), lambda i: (i, 0)),
    out_shape=jax.ShapeDtypeStruct((NPAD, D), jnp.float32),
)


def _tc_e_body(er_ref, w1_ref, b1_ref, w2_ref, b2_ref, out_ref):
    h = jnp.maximum(_bf16_dot(er_ref[...], w1_ref[...]) + b1_ref[...], 0.0)
    out_ref[...] = _bf16_dot(h, w2_ref[...]) + b2_ref[...]


BE = 8192

_tc_e = pl.pallas_call(
    _tc_e_body,
    grid=(EPAD // BE,),
    in_specs=[
        pl.BlockSpec((BE, D), lambda i: (i, 0)),
        pl.BlockSpec((D, D), lambda i: (0, 0)),
        pl.BlockSpec((1, D), lambda i: (0, 0)),
        pl.BlockSpec((D, 1), lambda i: (0, 0)),
        pl.BlockSpec((1, 1), lambda i: (0, 0)),
    ],
    out_specs=pl.BlockSpec((BE, 1), lambda i: (i, 0)),
    out_shape=jax.ShapeDtypeStruct((EPAD, 1), jnp.float32),
)


def kernel(x, edge_index, W1, b1, W2, b2, LW1, Lb1, LW2, Lb2):
    pad_e = IDXROWS * K - E
    rowp = jnp.concatenate([edge_index[0], jnp.zeros((pad_e,), jnp.int32)])
    colp = jnp.concatenate([edge_index[1], jnp.full((pad_e,), N, jnp.int32)])
    row2d = rowp.reshape(IDXROWS, K)
    col2d = colp.reshape(IDXROWS, K)
    xp = jnp.pad(x, ((0, NPAD - N), (0, 0)))
    zeros1 = jnp.zeros((NPAD,), jnp.float32)
    zeros2 = jnp.zeros((NPAD, D), jnp.float32)

    degp = _sc_degree(col2d, zeros1).reshape(2, NPAD, 1)
    hs1, dinv = _tc_a(xp, W1, degp)
    acc1 = _sc_scatter(hs1, row2d, col2d, zeros2).reshape(2, NPAD, D)
    hs2 = _tc_b(acc1, hs1, dinv, b1.reshape(1, D), W2)
    acc2 = _sc_scatter(hs2, row2d, col2d, zeros2).reshape(2, NPAD, D)
    z2 = _tc_c(acc2, hs2, dinv, b2.reshape(1, D))
    er = _sc_edge(z2, row2d, col2d)
    out = _tc_e(er, LW1, Lb1.reshape(1, D), LW2, Lb2.reshape(1, 1))
    return out.reshape(EPAD)[:E]


# speedup vs baseline: 6.4471x; 1.1335x over previous
import functools

import jax
import jax.numpy as jnp
from jax import lax
from jax.experimental import pallas as pl
from jax.experimental.pallas import tpu as pltpu
from jax.experimental.pallas import tpu_sc as plsc

N = 10000
E = 320000
D = 128

NC = 2
NS = 16
NW = NC * NS
K = 128

NPAD = ((N + NW * 8 - 1) // (NW * 8)) * NW * 8
RPS = NPAD // NS
CHUNKS = ((-(-E // (NW * K)) + 7) // 8) * 8
EPT = CHUNKS * K
EPAD = NW * EPT
NCHUNKS = NW * CHUNKS

CMAX = CHUNKS
IDXROWS = NCHUNKS

_mesh = plsc.VectorSubcoreMesh(core_axis_name="c", subcore_axis_name="s")


def _wid():
    return lax.axis_index("c") * NS + lax.axis_index("s")


@functools.partial(
    pl.kernel,
    out_type=jax.ShapeDtypeStruct((NC * NPAD,), jnp.float32),
    mesh=_mesh,
    scratch_types=[
        pltpu.VMEM_SHARED((NPAD,), jnp.float32),
        pltpu.VMEM((CHUNKS, K), jnp.int32),
        pltpu.VMEM((K,), jnp.float32),
    ],
)
def _sc_degree(col2d, zeros1, deg_out, acc, call, ones):
    core = lax.axis_index("c")
    sid = lax.axis_index("s")
    w = _wid()
    pltpu.sync_copy(zeros1.at[pl.ds(sid * RPS, RPS)], acc.at[pl.ds(sid * RPS, RPS)])
    pltpu.sync_copy(col2d.at[pl.ds(w * CHUNKS, CHUNKS)], call)
    for j in range(K // 16):
        ones[pl.ds(j * 16, 16)] = jnp.ones((16,), jnp.float32)
    plsc.subcore_barrier()

    def step(i, c):
        pltpu.sync_copy(ones, acc.at[call.at[i]], add=True)
        return c

    lax.fori_loop(0, CHUNKS, step, 0)
    plsc.subcore_barrier()
    pltpu.sync_copy(
        acc.at[pl.ds(sid * RPS, RPS)],
        deg_out.at[pl.ds(core * NPAD + sid * RPS, RPS)],
    )


@functools.partial(
    pl.kernel,
    out_type=jax.ShapeDtypeStruct((NC * NPAD, D), jnp.float32),
    mesh=_mesh,
    scratch_types=[
        pltpu.VMEM_SHARED((NPAD, D), jnp.float32),
        pltpu.VMEM((CMAX // 2, K), jnp.int32),
        pltpu.VMEM((CMAX // 2, K), jnp.int32),
        pltpu.VMEM((K, D), jnp.float32),
        pltpu.VMEM((K, D), jnp.float32),
        pltpu.SemaphoreType.DMA,
        pltpu.SemaphoreType.DMA,
    ],
)
def _sc_scatter(hs, row2d, col2d, zeros2, acc_out, acc, rall, call, msg0, msg1, sem0, sem1):
    core = lax.axis_index("c")
    sid = lax.axis_index("s")
    base = _wid() * CHUNKS
    hc = CHUNKS // 2
    pltpu.sync_copy(zeros2.at[pl.ds(sid * RPS, RPS)], acc.at[pl.ds(sid * RPS, RPS)])
    plsc.subcore_barrier()

    for h in range(2):
        off = base + h * hc
        pltpu.sync_copy(row2d.at[pl.ds(off, CMAX // 2)], rall)
        pltpu.sync_copy(col2d.at[pl.ds(off, CMAX // 2)], call)

        pltpu.async_copy(hs.at[rall.at[0]], msg0, sem0)

        def step(g, c):
            i0 = 2 * g
            i1 = i0 + 1
            i2 = jnp.minimum(i1 + 1, hc - 1)
            pltpu.make_async_copy(hs.at[rall.at[i0]], msg0, sem0).wait()
            pltpu.async_copy(hs.at[rall.at[i1]], msg1, sem1)
            pltpu.sync_copy(msg0, acc.at[call.at[i0]], add=True)
            pltpu.make_async_copy(hs.at[rall.at[i1]], msg1, sem1).wait()
            pltpu.async_copy(hs.at[rall.at[i2]], msg0, sem0)
            pltpu.sync_copy(msg1, acc.at[call.at[i1]], add=True)
            return c

        lax.fori_loop(0, hc // 2, step, 0)
        pltpu.make_async_copy(hs.at[rall.at[0]], msg0, sem0).wait()

    plsc.subcore_barrier()
    pltpu.sync_copy(
        acc.at[pl.ds(sid * RPS, RPS)],
        acc_out.at[pl.ds(core * NPAD + sid * RPS, RPS)],
    )


@functools.partial(
    pl.kernel,
    out_type=jax.ShapeDtypeStruct((EPAD, D), jnp.float32),
    mesh=_mesh,
    scratch_types=[
        pltpu.VMEM((CMAX, K), jnp.int32),
        pltpu.VMEM((CMAX, K), jnp.int32),
        pltpu.VMEM((K, D), jnp.float32),
        pltpu.VMEM((K, D), jnp.float32),
        pltpu.VMEM((K, D), jnp.float32),
        pltpu.VMEM((K, D), jnp.float32),
        pltpu.VMEM((K, D), jnp.float32),
        pltpu.VMEM((K, D), jnp.float32),
        pltpu.SemaphoreType.DMA,
        pltpu.SemaphoreType.DMA,
        pltpu.SemaphoreType.DMA,
        pltpu.SemaphoreType.DMA,
    ],
)
def _sc_edge(z, row2d, col2d, out, rall, call, bs0, bd0, bs1, bd1, ob0, ob1, semA, semB, semO0, semO1):
    base = _wid() * CHUNKS
    nch = CHUNKS
    ebase = base * K
    pltpu.sync_copy(row2d.at[pl.ds(base, CMAX)], rall)
    pltpu.sync_copy(col2d.at[pl.ds(base, CMAX)], call)

    def issue(i, bs, bd, sem):
        pltpu.async_copy(z.at[rall.at[i]], bs, sem)
        pltpu.async_copy(z.at[call.at[i]], bd, sem)

    def drain(bs, bd, sem):
        pltpu.make_async_copy(z.at[rall.at[0]], bs, sem).wait()
        pltpu.make_async_copy(z.at[call.at[0]], bd, sem).wait()

    def compute(i, bs, bd, obx, semO, g):
        @pl.when(g > 0)
        def _():
            pltpu.make_async_copy(obx, out.at[pl.ds(ebase, K)], semO).wait()

        def edge(e, c2):
            for j in range(D // 16):
                obx[e, pl.ds(j * 16, 16)] = (
                    bs[e, pl.ds(j * 16, 16)] + bd[e, pl.ds(j * 16, 16)]
                ) * 0.5
            return c2

        lax.fori_loop(0, K, edge, 0)
        pltpu.async_copy(obx, out.at[pl.ds(ebase + i * K, K)], semO)

    issue(0, bs0, bd0, semA)

    def chunk(g, c):
        i0 = 2 * g
        i1 = i0 + 1
        i2 = jnp.minimum(i1 + 1, nch - 1)
        drain(bs0, bd0, semA)
        issue(i1, bs1, bd1, semB)
        compute(i0, bs0, bd0, ob0, semO0, g)
        drain(bs1, bd1, semB)
        issue(i2, bs0, bd0, semA)
        compute(i1, bs1, bd1, ob1, semO1, g)
        return c

    lax.fori_loop(0, nch // 2, chunk, 0)
    drain(bs0, bd0, semA)
    pltpu.make_async_copy(ob0, out.at[pl.ds(ebase, K)], semO0).wait()
    pltpu.make_async_copy(ob1, out.at[pl.ds(ebase, K)], semO1).wait()


BR = 2560


def _bf16_dot(a, b):
    return jnp.dot(
        a.astype(jnp.bfloat16),
        b.astype(jnp.bfloat16),
        preferred_element_type=jnp.float32,
    )


def _dot3x(a, b):
    ah = a.astype(jnp.bfloat16)
    bh = b.astype(jnp.bfloat16)
    al = (a - ah.astype(jnp.float32)).astype(jnp.bfloat16)
    bl = (b - bh.astype(jnp.float32)).astype(jnp.bfloat16)

    def d(p, q):
        return jnp.dot(p, q, preferred_element_type=jnp.float32)

    return d(ah, bh) + (d(ah, bl) + d(al, bh))


def _tc_a_body(x_ref, w_ref, degp_ref, hs_ref, dinv_ref):
    deg = degp_ref[0] + degp_ref[1] + 1.0
    dinv = lax.rsqrt(deg)
    dinv_ref[...] = dinv
    hs_ref[...] = _bf16_dot(x_ref[...], w_ref[...]) * dinv


_tc_a = pl.pallas_call(
    _tc_a_body,
    grid=(NPAD // BR,),
    in_specs=[
        pl.BlockSpec((BR, D), lambda i: (i, 0)),
        pl.BlockSpec((D, D), lambda i: (0, 0)),
        pl.BlockSpec((2, BR, 1), lambda i: (0, i, 0)),
    ],
    out_specs=[
        pl.BlockSpec((BR, D), lambda i: (i, 0)),
        pl.BlockSpec((BR, 1), lambda i: (i, 0)),
    ],
    out_shape=[
        jax.ShapeDtypeStruct((NPAD, D), jnp.float32),
        jax.ShapeDtypeStruct((NPAD, 1), jnp.float32),
    ],
)


def _tc_b_body(acc_ref, hs_ref, dinv_ref, b_ref, w_ref, out_ref):
    z = (acc_ref[0] + acc_ref[1] + hs_ref[...]) * dinv_ref[...] + b_ref[...]
    z = jnp.maximum(z, 0.0)
    out_ref[...] = _bf16_dot(z, w_ref[...]) * dinv_ref[...]


_tc_b = pl.pallas_call(
    _tc_b_body,
    grid=(NPAD // BR,),
    in_specs=[
        pl.BlockSpec((2, BR, D), lambda i: (0, i, 0)),
        pl.BlockSpec((BR, D), lambda i: (i, 0)),
        pl.BlockSpec((BR, 1), lambda i: (i, 0)),
        pl.BlockSpec((1, D), lambda i: (0, 0)),
        pl.BlockSpec((D, D), lambda i: (0, 0)),
    ],
    out_specs=pl.BlockSpec((BR, D), lambda i: (i, 0)),
    out_shape=jax.ShapeDtypeStruct((NPAD, D), jnp.float32),
)


def _tc_c_body(acc_ref, hs_ref, dinv_ref, b_ref, z_ref):
    z_ref[...] = (acc_ref[0] + acc_ref[1] + hs_ref[...]) * dinv_ref[...] + b_ref[...]


_tc_c = pl.pallas_call(
    _tc_c_body,
    grid=(NPAD // BR,),
    in_specs=[
        pl.BlockSpec((2, BR, D), lambda i: (0, i, 0)),
        pl.BlockSpec((BR, D), lambda i: (i, 0)),
        pl.BlockSpec((BR, 1), lambda i: (i, 0)),
        pl.BlockSpec((1, D), lambda i: (0, 0)),
    ],
    out_specs=pl.BlockSpec((BR, D), lambda i: (i, 0)),
    out_shape=jax.ShapeDtypeStruct((NPAD, D), jnp.float32),
)


def _tc_e_body(er_ref, w1_ref, b1_ref, w2_ref, b2_ref, out_ref):
    h = jnp.maximum(_bf16_dot(er_ref[...], w1_ref[...]) + b1_ref[...], 0.0)
    out_ref[...] = _bf16_dot(h, w2_ref[...]) + b2_ref[...]


BE = 8192

_tc_e = pl.pallas_call(
    _tc_e_body,
    grid=(EPAD // BE,),
    in_specs=[
        pl.BlockSpec((BE, D), lambda i: (i, 0)),
        pl.BlockSpec((D, D), lambda i: (0, 0)),
        pl.BlockSpec((1, D), lambda i: (0, 0)),
        pl.BlockSpec((D, 1), lambda i: (0, 0)),
        pl.BlockSpec((1, 1), lambda i: (0, 0)),
    ],
    out_specs=pl.BlockSpec((BE, 1), lambda i: (i, 0)),
    out_shape=jax.ShapeDtypeStruct((EPAD, 1), jnp.float32),
)


def kernel(x, edge_index, W1, b1, W2, b2, LW1, Lb1, LW2, Lb2):
    pad_e = IDXROWS * K - E
    rowp = jnp.concatenate([edge_index[0], jnp.zeros((pad_e,), jnp.int32)])
    colp = jnp.concatenate([edge_index[1], jnp.full((pad_e,), N, jnp.int32)])
    row2d = rowp.reshape(IDXROWS, K)
    col2d = colp.reshape(IDXROWS, K)
    xp = jnp.pad(x, ((0, NPAD - N), (0, 0)))
    zeros1 = jnp.zeros((NPAD,), jnp.float32)
    zeros2 = jnp.zeros((NPAD, D), jnp.float32)

    degp = _sc_degree(col2d, zeros1).reshape(2, NPAD, 1)
    hs1, dinv = _tc_a(xp, W1, degp)
    acc1 = _sc_scatter(hs1, row2d, col2d, zeros2).reshape(2, NPAD, D)
    hs2 = _tc_b(acc1, hs1, dinv, b1.reshape(1, D), W2)
    acc2 = _sc_scatter(hs2, row2d, col2d, zeros2).reshape(2, NPAD, D)
    z2 = _tc_c(acc2, hs2, dinv, b2.reshape(1, D))
    er = _sc_edge(z2, row2d, col2d)
    out = _tc_e(er, LW1, Lb1.reshape(1, D), LW2, Lb2.reshape(1, 1))
    return out.reshape(EPAD)[:E]
